# Initial kernel scaffold; baseline (speedup 1.0000x reference)
import jax
import jax.numpy as jnp
from jax.experimental import pallas as pl


def kernel(inputs, embedding_weight):
    raise NotImplementedError("write your pallas kernel here")


# speedup vs baseline: 1.7936x; 1.7936x over previous
import functools

import jax
import jax.numpy as jnp
from jax.experimental import pallas as pl
from jax.experimental.pallas import tpu as pltpu
from jax.experimental.pallas import tpu_sc as plsc

_N = 8192
_D = 256
_R = 512
_C = 1024
_NI = _N // _R
_NJ = _N // _C


def _phase1_body(x_ref, e_ref, xsq_ref, esq_ref, idx_ref, loss_ref,
                 minv, mini, gram, sum_min, sum_e4):
    i = pl.program_id(0)
    j = pl.program_id(1)
    x = x_ref[...]
    e = e_ref[...]
    xe = jax.lax.dot_general(x, e, (((1,), (1,)), ((), ())),
                             preferred_element_type=jnp.float32)
    d = (xsq_ref[...] + esq_ref[...]) - 2.0 * xe

    tmin = jnp.min(d, axis=1, keepdims=True)
    cols = jax.lax.broadcasted_iota(jnp.int32, (_R, _C), 1) + j * _C
    targ = jnp.min(jnp.where(d == tmin, cols, jnp.int32(2 ** 30)),
                   axis=1, keepdims=True)

    @pl.when(j == 0)
    def _():
        minv[...] = tmin
        mini[...] = targ

    @pl.when(j > 0)
    def _():
        better = tmin < minv[...]
        mini[...] = jnp.where(better, targ, mini[...])
        minv[...] = jnp.where(better, tmin, minv[...])

    @pl.when(i == 0)
    def _():
        g = jax.lax.dot_general(e, e, (((0,), (0,)), ((), ())),
                                preferred_element_type=jnp.float32)
        esq_v = esq_ref[...]
        s4 = jnp.sum(esq_v * esq_v)

        @pl.when(j == 0)
        def _():
            gram[...] = g
            sum_e4[0, 0] = s4

        @pl.when(j > 0)
        def _():
            gram[...] += g
            sum_e4[0, 0] += s4

    @pl.when(j == _NJ - 1)
    def _():
        idx_ref[...] = mini[...]
        row_sum = jnp.sum(minv[...])

        @pl.when(i == 0)
        def _():
            sum_min[0, 0] = row_sum

        @pl.when(i > 0)
        def _():
            sum_min[0, 0] += row_sum

    @pl.when((i == _NI - 1) & (j == _NJ - 1))
    def _():
        m = gram[...]
        ortho_sq = jnp.sum(m * m) - sum_e4[0, 0]
        ortho = jnp.sqrt(jnp.maximum(ortho_sq, 0.0))
        mse = sum_min[0, 0] / jnp.float32(_N * _D)
        loss_ref[...] = jnp.full((1, 1), mse + 0.25 * mse + 0.09 * ortho,
                                 jnp.float32)


def _make_phase1(interpret=False):
    return pl.pallas_call(
        _phase1_body,
        grid=(_NI, _NJ),
        in_specs=[
            pl.BlockSpec((_R, _D), lambda i, j: (i, 0)),
            pl.BlockSpec((_C, _D), lambda i, j: (j, 0)),
            pl.BlockSpec((_R, 1), lambda i, j: (i, 0)),
            pl.BlockSpec((1, _C), lambda i, j: (0, j)),
        ],
        out_specs=[
            pl.BlockSpec((_R, 1), lambda i, j: (i, 0)),
            pl.BlockSpec((1, 1), lambda i, j: (0, 0)),
        ],
        out_shape=[
            jax.ShapeDtypeStruct((_N, 1), jnp.int32),
            jax.ShapeDtypeStruct((1, 1), jnp.float32),
        ],
        scratch_shapes=[
            pltpu.VMEM((_R, 1), jnp.float32),
            pltpu.VMEM((_R, 1), jnp.int32),
            pltpu.VMEM((_D, _D), jnp.float32),
            pltpu.SMEM((1, 1), jnp.float32),
            pltpu.SMEM((1, 1), jnp.float32),
        ],
        compiler_params=pltpu.CompilerParams(
            dimension_semantics=("arbitrary", "arbitrary")),
        interpret=interpret,
    )


def _ew_body(x_ref, q_ref, o_ref):
    o_ref[...] = x_ref[...] + (q_ref[...] - x_ref[...])


def _make_ew(interpret=False):
    return pl.pallas_call(
        _ew_body,
        grid=(8,),
        in_specs=[
            pl.BlockSpec((1024, _D), lambda i: (i, 0)),
            pl.BlockSpec((1024, _D), lambda i: (i, 0)),
        ],
        out_specs=pl.BlockSpec((1024, _D), lambda i: (i, 0)),
        out_shape=jax.ShapeDtypeStruct((_N, _D), jnp.float32),
        interpret=interpret,
    )


def _sc_gather(table, idx):
    mesh = plsc.VectorSubcoreMesh(core_axis_name="c", subcore_axis_name="s")
    n_workers = 32
    bpw = _N // n_workers

    @functools.partial(
        pl.kernel,
        out_type=jax.ShapeDtypeStruct((_N, _D), jnp.float32),
        mesh=mesh,
        scratch_types=[
            pltpu.VMEM((bpw,), jnp.int32),
            pltpu.VMEM((bpw, _D), jnp.float32),
            pltpu.SemaphoreType.DMA,
        ],
    )
    def gather_kernel(table_hbm, idx_hbm, out_hbm, idx_v, rows_v, sem):
        wid = jax.lax.axis_index("s") * 2 + jax.lax.axis_index("c")
        base = wid * bpw
        pltpu.sync_copy(idx_hbm.at[pl.ds(base, bpw)], idx_v)
        pltpu.async_copy(table_hbm.at[idx_v], rows_v, sem).wait()
        pltpu.sync_copy(rows_v, out_hbm.at[pl.ds(base, bpw)])

    return gather_kernel(table, idx)


def kernel(inputs, embedding_weight):
    input_shape = inputs.shape
    x = inputs.reshape(-1, _D)
    xsq = jnp.sum(x ** 2, axis=1, keepdims=True)
    esq = jnp.sum(embedding_weight ** 2, axis=1)

    idx2d, loss11 = _make_phase1()(x, embedding_weight, xsq,
                                   esq.reshape(1, _N))
    q = _sc_gather(embedding_weight, idx2d.reshape(_N))
    quantized_st = _make_ew()(x, q)
    return (quantized_st.reshape(input_shape), loss11[0, 0], idx2d, inputs)
